# baseline (speedup 1.0000x reference)
import jax
import jax.numpy as jnp
from jax.experimental import pallas as pl
from jax.experimental.pallas import tpu as pltpu

_TOPK = 100
_STRIDE = 4.0
_MIN_SCORE = 0.05
_C, _H, _W = 80, 128, 128
_CPB = 16
_G = 8
_SPI = _C // _CPB


def _nms_scores(hm):
    s = jax.nn.sigmoid(hm)
    ninf_row = jnp.full((1, _W), -jnp.inf, jnp.float32)
    up = jnp.concatenate([s[1:, :], ninf_row], axis=0)
    dn = jnp.concatenate([ninf_row, s[:-1, :]], axis=0)
    mv = jnp.maximum(s, jnp.maximum(up, dn))
    ninf_col = jnp.full((_H, 1), -jnp.inf, jnp.float32)
    lf = jnp.concatenate([mv[:, 1:], ninf_col], axis=1)
    rt = jnp.concatenate([ninf_col, mv[:, :-1]], axis=1)
    mh = jnp.maximum(mv, jnp.maximum(lf, rt))
    return jnp.where(mh == s, s, 0.0)


def _decoder_kernel(hm_ref, fhi_ref, fmid_ref, flo_ref,
                    score_out_ref, cls_out_ref, bbox_out_ref,
                    rowmax_ref, *sc_refs):
    step = pl.program_id(1)
    img = step // _SPI
    cb = step % _SPI

    rms = []
    tiles = []
    for ci in range(_CPB):
        score = _nms_scores(hm_ref[0, ci])
        tiles.append(score.reshape(_H // 8, 8, _W))
        rms.append(jnp.max(score.T, axis=0, keepdims=True))
    for i in range(_G):
        @pl.when(img == i)
        def _store(i=i):
            for ci in range(_CPB):
                sc_refs[i][pl.ds((cb * _CPB + ci) * (_H // 8), _H // 8)] = (
                    tiles[ci])
    rowmax_ref[pl.ds(img * 10 + cb * 2, 1)] = (
        jnp.concatenate(rms[:8], axis=0)[None])
    rowmax_ref[pl.ds(img * 10 + cb * 2 + 1, 1)] = (
        jnp.concatenate(rms[8:], axis=0)[None])

    @pl.when(step == _G * _SPI - 1)
    def _select():
        lane1 = jax.lax.broadcasted_iota(jnp.int32, (1, _W), 1)
        sub3 = jax.lax.broadcasted_iota(jnp.int32, (1, 8, 128), 1)
        lane3 = jax.lax.broadcasted_iota(jnp.int32, (1, 8, 128), 2)
        d0 = jax.lax.broadcasted_iota(jnp.int32, (10, 8, 128), 0)
        d1 = jax.lax.broadcasted_iota(jnp.int32, (10, 8, 128), 1)
        d2 = jax.lax.broadcasted_iota(jnp.int32, (10, 8, 128), 2)
        r_iota = d0 * 1024 + d1 * 128 + d2

        def pick_one(i, m, r, rowmax_i):
            t = r // 8
            sub = r % 8
            tile = sc_refs[i][pl.ds(t, 1)]
            hit = (sub3 == sub) & (tile == m)
            w = jnp.min(jnp.where(hit, lane3, 128))
            sel = (sub3 == sub) & (lane3 == w)
            sc_refs[i][pl.ds(t, 1)] = jnp.where(sel, -1.0, tile)
            nrm = jnp.max(jnp.where(sub3 == sub,
                                    jnp.where(lane3 == w, -1.0, tile), -1.0))
            rm_excl = jnp.where(r_iota == r, -1.0, rowmax_i)
            mx2 = jnp.max(rm_excl)
            r2 = jnp.min(jnp.where(rm_excl == mx2, r_iota, 10240))
            rowmax_i = jnp.where(r_iota == r, nrm, rowmax_i)
            nm = jnp.maximum(nrm, mx2)
            nr = jnp.where(nrm > mx2, r,
                           jnp.where(nrm < mx2, r2, jnp.minimum(r, r2)))
            return w, nm, nr, rowmax_i

        def body(k, carry):
            kmask = lane1 == k
            out = []
            for i in range(_G):
                rowmax_i, m, r, rv, rr, rw = carry[i]
                w, nm, nr, rowmax_i = pick_one(i, m, r, rowmax_i)
                rv = jnp.where(kmask, m, rv)
                rr = jnp.where(kmask, r, rr)
                rw = jnp.where(kmask, w, rw)
                out.append((rowmax_i, nm, nr, rv, rr, rw))
            return tuple(out)

        zf = jnp.zeros((1, _W), jnp.float32)
        zi = jnp.zeros((1, _W), jnp.int32)
        init = []
        for i in range(_G):
            rowmax0 = rowmax_ref[i * 10:(i + 1) * 10]
            m0 = jnp.max(rowmax0)
            r0 = jnp.min(jnp.where(rowmax0 == m0, r_iota, 10240))
            init.append((rowmax0, m0, r0, zf, zi, zi))
        fin = jax.lax.fori_loop(0, _TOPK, body, tuple(init))

        lane2 = jax.lax.broadcasted_iota(jnp.int32, (128, 128), 1)
        for i in range(_G):
            _, _, _, rv, rr, rw = fin[i]
            cc = rr // _H
            hh = rr % _H
            hh_t = jnp.transpose(hh.reshape(1, 128), (1, 0))
            ww_t = jnp.transpose(rw.reshape(1, 128), (1, 0))
            x_oh = jnp.where(hh_t == lane2, 1.0, 0.0).astype(jnp.bfloat16)
            y_oh = jnp.where(ww_t == lane2, 1.0, 0.0)
            picked = 0.0
            for part_ref in (fhi_ref, fmid_ref, flo_ref):
                rows = jax.lax.dot_general(
                    x_oh, part_ref[i], (((1,), (0,)), ((), ())),
                    preferred_element_type=jnp.float32)
                picked = picked + jnp.sum(
                    rows.reshape(128, 4, 128) * y_oh.reshape(128, 1, 128),
                    axis=2)
            gath = jnp.transpose(picked, (1, 0))

            xs = rw.astype(jnp.float32) + gath[0:1, :]
            ys = hh.astype(jnp.float32) + gath[1:2, :]
            bw = gath[2:3, :]
            bh = gath[3:4, :]
            mask = rv > _MIN_SCORE
            score_out_ref[i] = jnp.where(mask, rv, -1.0)
            cls_out_ref[i] = jnp.where(mask, cc.astype(jnp.float32), -1.0)
            x1 = jnp.where(mask, (xs - bw * 0.5) * _STRIDE, 0.0)
            y1 = jnp.where(mask, (ys - bh * 0.5) * _STRIDE, 0.0)
            x2 = jnp.where(mask, (xs + bw * 0.5) * _STRIDE, 0.0)
            y2 = jnp.where(mask, (ys + bh * 0.5) * _STRIDE, 0.0)
            bbox_out_ref[i] = jnp.concatenate([x1, y1, x2, y2], axis=0)


def kernel(heatmap_heads, offset_heads, wh_heads):
    B = heatmap_heads.shape[0]
    fmat = jnp.concatenate(
        [offset_heads[:, 0], offset_heads[:, 1],
         wh_heads[:, 0], wh_heads[:, 1]], axis=2)
    f_hi = jax.lax.optimization_barrier(fmat.astype(jnp.bfloat16))
    res1 = fmat - f_hi.astype(jnp.float32)
    f_mid = jax.lax.optimization_barrier(res1.astype(jnp.bfloat16))
    f_lo = (res1 - f_mid.astype(jnp.float32)).astype(jnp.bfloat16)
    out_shape = (
        jax.ShapeDtypeStruct((B, 1, 128), jnp.float32),
        jax.ShapeDtypeStruct((B, 1, 128), jnp.float32),
        jax.ShapeDtypeStruct((B, 4, 128), jnp.float32),
    )
    scores, classes, bbox = pl.pallas_call(
        _decoder_kernel,
        grid=(B // _G, _G * _SPI),
        in_specs=[
            pl.BlockSpec((1, _CPB, _H, _W),
                         lambda g, s: (g * _G + s // _SPI, s % _SPI, 0, 0)),
            pl.BlockSpec((_G, _H, 4 * _W), lambda g, s: (g, 0, 0)),
            pl.BlockSpec((_G, _H, 4 * _W), lambda g, s: (g, 0, 0)),
            pl.BlockSpec((_G, _H, 4 * _W), lambda g, s: (g, 0, 0)),
        ],
        out_specs=(
            pl.BlockSpec((_G, 1, 128), lambda g, s: (g, 0, 0)),
            pl.BlockSpec((_G, 1, 128), lambda g, s: (g, 0, 0)),
            pl.BlockSpec((_G, 4, 128), lambda g, s: (g, 0, 0)),
        ),
        out_shape=out_shape,
        scratch_shapes=(
            [pltpu.VMEM((_G * 10, 8, 128), jnp.float32)] +
            [pltpu.VMEM((_C * _H // 8, 8, _W), jnp.float32)
             for _ in range(_G)]),
    )(heatmap_heads, f_hi, f_mid, f_lo)
    return (scores[:, 0, :_TOPK],
            classes[:, 0, :_TOPK],
            bbox.transpose(0, 2, 1)[:, :_TOPK, :])


# speedup vs baseline: 1.0372x; 1.0372x over previous
import jax
import jax.numpy as jnp
from jax.experimental import pallas as pl
from jax.experimental.pallas import tpu as pltpu

_TOPK = 100
_STRIDE = 4.0
_MIN_SCORE = 0.05
_C, _H, _W = 80, 128, 128
_CPB = 16
_G = 8
_SPI = _C // _CPB


def _nms_scores(hm):
    s = jax.nn.sigmoid(hm)
    ninf_row = jnp.full((1, _W), -jnp.inf, jnp.float32)
    up = jnp.concatenate([s[1:, :], ninf_row], axis=0)
    dn = jnp.concatenate([ninf_row, s[:-1, :]], axis=0)
    mv = jnp.maximum(s, jnp.maximum(up, dn))
    ninf_col = jnp.full((_H, 1), -jnp.inf, jnp.float32)
    lf = jnp.concatenate([mv[:, 1:], ninf_col], axis=1)
    rt = jnp.concatenate([ninf_col, mv[:, :-1]], axis=1)
    mh = jnp.maximum(mv, jnp.maximum(lf, rt))
    return jnp.where(mh == s, s, 0.0)


def _decoder_kernel(hm_ref, fhi_ref, fmid_ref, flo_ref,
                    score_out_ref, cls_out_ref, bbox_out_ref,
                    rowmax_ref, *sc_refs):
    step = pl.program_id(1)
    img = step // _SPI
    cb = step % _SPI

    rms = []
    tiles = []
    for ci in range(_CPB):
        score = _nms_scores(hm_ref[0, ci])
        tiles.append(score.reshape(_H // 8, 8, _W))
        rms.append(jnp.max(score.T, axis=0, keepdims=True))
    for i in range(_G):
        @pl.when(img == i)
        def _store(i=i):
            for ci in range(_CPB):
                sc_refs[i][pl.ds((cb * _CPB + ci) * (_H // 8), _H // 8)] = (
                    tiles[ci])
    rowmax_ref[pl.ds(img * 10 + cb * 2, 1)] = (
        jnp.concatenate(rms[:8], axis=0)[None])
    rowmax_ref[pl.ds(img * 10 + cb * 2 + 1, 1)] = (
        jnp.concatenate(rms[8:], axis=0)[None])

    @pl.when(step == _G * _SPI - 1)
    def _select():
        lane1 = jax.lax.broadcasted_iota(jnp.int32, (1, _W), 1)
        sub3 = jax.lax.broadcasted_iota(jnp.int32, (1, 8, 128), 1)
        lane3 = jax.lax.broadcasted_iota(jnp.int32, (1, 8, 128), 2)
        d0 = jax.lax.broadcasted_iota(jnp.int32, (10, 8, 128), 0)
        d1 = jax.lax.broadcasted_iota(jnp.int32, (10, 8, 128), 1)
        d2 = jax.lax.broadcasted_iota(jnp.int32, (10, 8, 128), 2)
        r_iota = d0 * 1024 + d1 * 128 + d2

        def pick_one(i, m, r, rowmax_i):
            t = r // 8
            sub = r % 8
            tile = sc_refs[i][pl.ds(t, 1)]
            hit = (sub3 == sub) & (tile == m)
            w = jnp.min(jnp.where(hit, lane3, 128))
            sel = (sub3 == sub) & (lane3 == w)
            sc_refs[i][pl.ds(t, 1)] = jnp.where(sel, -1.0, tile)
            nrm = jnp.max(jnp.where(sub3 == sub,
                                    jnp.where(lane3 == w, -1.0, tile), -1.0))
            rm_excl = jnp.where(r_iota == r, -1.0, rowmax_i)
            mx2 = jnp.max(rm_excl)
            r2 = jnp.min(jnp.where(rm_excl == mx2, r_iota, 10240))
            rowmax_i = jnp.where(r_iota == r, nrm, rowmax_i)
            nm = jnp.maximum(nrm, mx2)
            nr = jnp.where(nrm > mx2, r,
                           jnp.where(nrm < mx2, r2, jnp.minimum(r, r2)))
            return w, nm, nr, rowmax_i

        def body(k, carry):
            kmask = lane1 == k
            out = []
            for i in range(_G):
                rowmax_i, m, r, rv, rr, rw = carry[i]
                w, nm, nr, rowmax_i = pick_one(i, m, r, rowmax_i)
                rv = jnp.where(kmask, m, rv)
                rr = jnp.where(kmask, r, rr)
                rw = jnp.where(kmask, w, rw)
                out.append((rowmax_i, nm, nr, rv, rr, rw))
            return tuple(out)

        zf = jnp.zeros((1, _W), jnp.float32)
        zi = jnp.zeros((1, _W), jnp.int32)
        init = []
        for i in range(_G):
            rowmax0 = rowmax_ref[i * 10:(i + 1) * 10]
            m0 = jnp.max(rowmax0)
            r0 = jnp.min(jnp.where(rowmax0 == m0, r_iota, 10240))
            init.append((rowmax0, m0, r0, zf, zi, zi))
        fin = jax.lax.fori_loop(0, _TOPK, body, tuple(init), unroll=2)

        lane2 = jax.lax.broadcasted_iota(jnp.int32, (128, 128), 1)
        for i in range(_G):
            _, _, _, rv, rr, rw = fin[i]
            cc = rr // _H
            hh = rr % _H
            hh_t = jnp.transpose(hh.reshape(1, 128), (1, 0))
            ww_t = jnp.transpose(rw.reshape(1, 128), (1, 0))
            x_oh = jnp.where(hh_t == lane2, 1.0, 0.0).astype(jnp.bfloat16)
            y_oh = jnp.where(ww_t == lane2, 1.0, 0.0)
            picked = 0.0
            for part_ref in (fhi_ref, fmid_ref, flo_ref):
                rows = jax.lax.dot_general(
                    x_oh, part_ref[i], (((1,), (0,)), ((), ())),
                    preferred_element_type=jnp.float32)
                picked = picked + jnp.sum(
                    rows.reshape(128, 4, 128) * y_oh.reshape(128, 1, 128),
                    axis=2)
            gath = jnp.transpose(picked, (1, 0))

            xs = rw.astype(jnp.float32) + gath[0:1, :]
            ys = hh.astype(jnp.float32) + gath[1:2, :]
            bw = gath[2:3, :]
            bh = gath[3:4, :]
            mask = rv > _MIN_SCORE
            score_out_ref[i] = jnp.where(mask, rv, -1.0)
            cls_out_ref[i] = jnp.where(mask, cc.astype(jnp.float32), -1.0)
            x1 = jnp.where(mask, (xs - bw * 0.5) * _STRIDE, 0.0)
            y1 = jnp.where(mask, (ys - bh * 0.5) * _STRIDE, 0.0)
            x2 = jnp.where(mask, (xs + bw * 0.5) * _STRIDE, 0.0)
            y2 = jnp.where(mask, (ys + bh * 0.5) * _STRIDE, 0.0)
            bbox_out_ref[i] = jnp.concatenate([x1, y1, x2, y2], axis=0)


def kernel(heatmap_heads, offset_heads, wh_heads):
    B = heatmap_heads.shape[0]
    fmat = jnp.concatenate(
        [offset_heads[:, 0], offset_heads[:, 1],
         wh_heads[:, 0], wh_heads[:, 1]], axis=2)
    f_hi = jax.lax.optimization_barrier(fmat.astype(jnp.bfloat16))
    res1 = fmat - f_hi.astype(jnp.float32)
    f_mid = jax.lax.optimization_barrier(res1.astype(jnp.bfloat16))
    f_lo = (res1 - f_mid.astype(jnp.float32)).astype(jnp.bfloat16)
    out_shape = (
        jax.ShapeDtypeStruct((B, 1, 128), jnp.float32),
        jax.ShapeDtypeStruct((B, 1, 128), jnp.float32),
        jax.ShapeDtypeStruct((B, 4, 128), jnp.float32),
    )
    scores, classes, bbox = pl.pallas_call(
        _decoder_kernel,
        grid=(B // _G, _G * _SPI),
        in_specs=[
            pl.BlockSpec((1, _CPB, _H, _W),
                         lambda g, s: (g * _G + s // _SPI, s % _SPI, 0, 0)),
            pl.BlockSpec((_G, _H, 4 * _W), lambda g, s: (g, 0, 0)),
            pl.BlockSpec((_G, _H, 4 * _W), lambda g, s: (g, 0, 0)),
            pl.BlockSpec((_G, _H, 4 * _W), lambda g, s: (g, 0, 0)),
        ],
        out_specs=(
            pl.BlockSpec((_G, 1, 128), lambda g, s: (g, 0, 0)),
            pl.BlockSpec((_G, 1, 128), lambda g, s: (g, 0, 0)),
            pl.BlockSpec((_G, 4, 128), lambda g, s: (g, 0, 0)),
        ),
        out_shape=out_shape,
        scratch_shapes=(
            [pltpu.VMEM((_G * 10, 8, 128), jnp.float32)] +
            [pltpu.VMEM((_C * _H // 8, 8, _W), jnp.float32)
             for _ in range(_G)]),
    )(heatmap_heads, f_hi, f_mid, f_lo)
    return (scores[:, 0, :_TOPK],
            classes[:, 0, :_TOPK],
            bbox.transpose(0, 2, 1)[:, :_TOPK, :])
